# Initial kernel scaffold; baseline (speedup 1.0000x reference)
import jax
import jax.numpy as jnp
from jax.experimental import pallas as pl


def kernel(features, regime, W1, b1, W2, b2, gamma, beta):
    raise NotImplementedError("write your pallas kernel here")


# speedup vs baseline: 2.4227x; 2.4227x over previous
import functools

import jax
import jax.numpy as jnp
from jax.experimental import pallas as pl
from jax.experimental.pallas import tpu as pltpu

B, N, D, R = 16, 64, 256, 8
BT = 1024


def _fused_kernel(regime_ref, x_ref, w1_ref, w2_ref, vecs_ref, out_ref):
    x = x_ref[0]
    w1 = w1_ref[0]
    w2 = w2_ref[0]
    b1 = vecs_ref[0, 0]
    b2 = vecs_ref[0, 1]
    g = vecs_ref[0, 2]
    bt = vecs_ref[0, 3]

    h = jnp.dot(x, w1, preferred_element_type=jnp.float32) + b1[None, :]
    h = 0.5 * h * (1.0 + jax.lax.erf(h * 0.7071067811865476))
    h = jnp.dot(h, w2, preferred_element_type=jnp.float32) + b2[None, :]

    mu = jnp.mean(h, axis=-1, keepdims=True)
    var = jnp.mean(h * h, axis=-1, keepdims=True) - mu * mu
    h = (h - mu) * jax.lax.rsqrt(var + 1e-5)
    out_ref[0] = h * g[None, :] + bt[None, :]


@jax.jit
def kernel(features, regime, W1, b1, W2, b2, gamma, beta):
    T = N * N
    x = features.reshape(B, T, D)
    w1t = W1.transpose(0, 2, 1)
    w2t = W2.transpose(0, 2, 1)
    vecs = jnp.stack([b1, b2, gamma, beta], axis=1)

    grid = (B, T // BT)

    out = pl.pallas_call(
        _fused_kernel,
        grid_spec=pltpu.PrefetchScalarGridSpec(
            num_scalar_prefetch=1,
            grid=grid,
            in_specs=[
                pl.BlockSpec((1, BT, D), lambda b, t, reg: (b, t, 0)),
                pl.BlockSpec((1, D, D), lambda b, t, reg: (reg[b], 0, 0)),
                pl.BlockSpec((1, D, D), lambda b, t, reg: (reg[b], 0, 0)),
                pl.BlockSpec((1, 4, D), lambda b, t, reg: (reg[b], 0, 0)),
            ],
            out_specs=pl.BlockSpec((1, BT, D), lambda b, t, reg: (b, t, 0)),
        ),
        out_shape=jax.ShapeDtypeStruct((B, T, D), jnp.float32),
    )(regime, x, w1t, w2t, vecs)

    return out.reshape(B, N, N, D)
